# baseline (speedup 1.0000x reference)
import jax
import jax.numpy as jnp
import numpy as np
from jax import lax
from jax.experimental import pallas as pl
from jax.experimental.pallas import tpu as pltpu
from jax.experimental.pallas import tpu_sc as plsc

D_MODEL = 64
MAX_LEN = 256
SEQ = 200

NW = 32
GRP = 128
BQ = 256
NQ = 4
NLANE = 16
SCALE = float(np.sqrt(np.float32(D_MODEL)))


def _pe_np():
    pos = np.arange(MAX_LEN)[:, np.newaxis]
    i = np.arange(D_MODEL)[np.newaxis, :]
    angle_rates = 1 / np.power(10000, 2 * (i // 2) / np.float32(D_MODEL))
    angle_rads = pos * angle_rates
    pe = np.zeros((MAX_LEN, D_MODEL), dtype=np.float32)
    pe[:, 0::2] = np.sin(angle_rads[:, 0::2])
    pe[:, 1::2] = np.cos(angle_rads[:, 1::2])
    return pe[:SEQ]


def _sc_body(table_hbm, idx_hbm, pe_hbm, out_hbm, idx_v, pe_v, buf, obuf, sem):
    wid = lax.axis_index("s") * 2 + lax.axis_index("c")
    units_per_w = (SEQ * NQ) // NW
    grps_per_unit = BQ // GRP

    pltpu.sync_copy(idx_hbm.at[wid], idx_v)
    pltpu.sync_copy(pe_hbm, pe_v)

    drow = [lax.iota(jnp.int32, NLANE) + db * NLANE for db in range(D_MODEL // NLANE)]

    def unit_body(i, carry):
        u = wid * units_per_w + i
        s = u // NQ
        q = lax.rem(u, NQ)

        copies = []
        for g in range(grps_per_unit):
            copies.append(pltpu.async_copy(
                table_hbm.at[idx_v.at[i * grps_per_unit + g]],
                buf.at[pl.ds(g * GRP, GRP)],
                sem,
            ))
        for cp in copies:
            cp.wait()

        pe_s = [pe_v[s, pl.ds(db * NLANE, NLANE)] for db in range(D_MODEL // NLANE)]

        def j_body(j, carry2):
            col = jnp.full((NLANE,), j, dtype=jnp.int32)
            for db in range(D_MODEL // NLANE):
                v = buf[j, pl.ds(db * NLANE, NLANE)]
                plsc.store_scatter(obuf, [drow[db], col], v * SCALE + pe_s[db])
            return carry2

        lax.fori_loop(0, BQ, j_body, 0)

        pltpu.sync_copy(obuf, out_hbm.at[s, :, pl.ds(q * BQ, BQ)])
        return carry

    lax.fori_loop(0, units_per_w, unit_body, 0)


def kernel(x, table):
    batch, seq = x.shape
    n = batch * seq
    xq = jnp.transpose(x).reshape(NW, n // (NW * GRP), GRP)
    pe = jnp.asarray(_pe_np())

    mesh = plsc.VectorSubcoreMesh(core_axis_name="c", subcore_axis_name="s")
    run = pl.kernel(
        _sc_body,
        mesh=mesh,
        out_type=jax.ShapeDtypeStruct((seq, D_MODEL, batch), jnp.float32),
        scratch_types=[
            pltpu.VMEM((n // (NW * GRP), GRP), jnp.int32),
            pltpu.VMEM((SEQ, D_MODEL), jnp.float32),
            pltpu.VMEM((BQ, D_MODEL), jnp.float32),
            pltpu.VMEM((D_MODEL, BQ), jnp.float32),
            pltpu.SemaphoreType.DMA,
        ],
        compiler_params=pltpu.CompilerParams(
            use_tc_tiling_on_sc=False, needs_layout_passes=False),
    )
    out = run(table, xq, pe)
    return jnp.transpose(out, (2, 0, 1))


# speedup vs baseline: 1.0492x; 1.0492x over previous
import jax
import jax.numpy as jnp
import numpy as np
from jax import lax
from jax.experimental import pallas as pl
from jax.experimental.pallas import tpu as pltpu
from jax.experimental.pallas import tpu_sc as plsc

D_MODEL = 64
MAX_LEN = 256
SEQ = 200

NW = 32
GRP = 128
BQ = 256
NQ = 4
NLANE = 16
SCALE = float(np.sqrt(np.float32(D_MODEL)))


def _pe_np():
    pos = np.arange(MAX_LEN)[:, np.newaxis]
    i = np.arange(D_MODEL)[np.newaxis, :]
    angle_rates = 1 / np.power(10000, 2 * (i // 2) / np.float32(D_MODEL))
    angle_rads = pos * angle_rates
    pe = np.zeros((MAX_LEN, D_MODEL), dtype=np.float32)
    pe[:, 0::2] = np.sin(angle_rads[:, 0::2])
    pe[:, 1::2] = np.cos(angle_rads[:, 1::2])
    return pe[:SEQ]


NBUF_G = 3
NBUF_O = 2


def _sc_body(table_hbm, idx_hbm, pe_hbm, out_hbm, idx_v, pe_v,
             bufs, obufs, sems_g, sems_w):
    wid = lax.axis_index("s") * 2 + lax.axis_index("c")
    units_per_w = (SEQ * NQ) // NW
    grps_per_unit = BQ // GRP

    pltpu.sync_copy(idx_hbm.at[wid], idx_v)
    pltpu.sync_copy(pe_hbm, pe_v)

    drow = [lax.iota(jnp.int32, NLANE) + db * NLANE for db in range(D_MODEL // NLANE)]

    def fire_gather(i):
        slot = i % NBUF_G
        for g in range(grps_per_unit):
            pltpu.async_copy(
                table_hbm.at[idx_v.at[i * grps_per_unit + g]],
                bufs[slot].at[pl.ds(g * GRP, GRP)],
                sems_g[slot],
            )

    def wait_gather(i):
        slot = i % NBUF_G
        for g in range(grps_per_unit):
            pltpu.make_async_copy(
                table_hbm.at[idx_v.at[g]],
                bufs[slot].at[pl.ds(g * GRP, GRP)],
                sems_g[slot],
            ).wait()

    def unit_sq(i):
        u = wid * units_per_w + i
        return u // NQ, lax.rem(u, NQ)

    def wait_write(i):
        s, q = unit_sq(i)
        pltpu.make_async_copy(
            obufs[i % NBUF_O],
            out_hbm.at[s, :, pl.ds(q * BQ, BQ)],
            sems_w[i % NBUF_O],
        ).wait()

    for i in range(min(NBUF_G, units_per_w)):
        fire_gather(i)

    for i in range(units_per_w):
        s, q = unit_sq(i)
        buf = bufs[i % NBUF_G]
        obuf = obufs[i % NBUF_O]

        wait_gather(i)
        if i >= NBUF_O:
            wait_write(i - NBUF_O)

        pe_s = [pe_v[s, pl.ds(db * NLANE, NLANE)] for db in range(D_MODEL // NLANE)]

        def j_body(j, carry2):
            col = jnp.full((NLANE,), j, dtype=jnp.int32)
            for db in range(D_MODEL // NLANE):
                v = buf[j, pl.ds(db * NLANE, NLANE)]
                plsc.store_scatter(obuf, [drow[db], col], v * SCALE + pe_s[db])
            return carry2

        lax.fori_loop(0, BQ, j_body, 0)

        pltpu.async_copy(obuf, out_hbm.at[s, :, pl.ds(q * BQ, BQ)], sems_w[i % NBUF_O])
        if i + NBUF_G < units_per_w:
            fire_gather(i + NBUF_G)

    for i in range(units_per_w - NBUF_O, units_per_w):
        wait_write(i)


def kernel(x, table):
    batch, seq = x.shape
    n = batch * seq
    xq = jnp.transpose(x).reshape(NW, n // (NW * GRP), GRP)
    pe = jnp.asarray(_pe_np())

    mesh = plsc.VectorSubcoreMesh(core_axis_name="c", subcore_axis_name="s")
    run = pl.kernel(
        _sc_body,
        mesh=mesh,
        out_type=jax.ShapeDtypeStruct((seq, D_MODEL, batch), jnp.float32),
        scratch_types=[
            pltpu.VMEM((n // (NW * GRP), GRP), jnp.int32),
            pltpu.VMEM((SEQ, D_MODEL), jnp.float32),
            [pltpu.VMEM((BQ, D_MODEL), jnp.float32) for _ in range(NBUF_G)],
            [pltpu.VMEM((D_MODEL, BQ), jnp.float32) for _ in range(NBUF_O)],
            [pltpu.SemaphoreType.DMA for _ in range(NBUF_G)],
            [pltpu.SemaphoreType.DMA for _ in range(NBUF_O)],
        ],
        compiler_params=pltpu.CompilerParams(
            use_tc_tiling_on_sc=False, needs_layout_passes=False),
    )
    out = run(table, xq, pe)
    return jnp.transpose(out, (2, 0, 1))
